# Initial kernel scaffold; baseline (speedup 1.0000x reference)
import jax
import jax.numpy as jnp
from jax.experimental import pallas as pl


def kernel(x, edge_index, W1, b1, W2, b2):
    raise NotImplementedError("write your pallas kernel here")


# speedup vs baseline: 29.0140x; 29.0140x over previous
import functools

import jax
import jax.numpy as jnp
from jax import lax
from jax.experimental import pallas as pl
from jax.experimental.pallas import tpu as pltpu
from jax.experimental.pallas import tpu_sc as plsc

N = 10000
NP = 10240
D_IN = 128
HID = 256
NCLS = 16
E = 320000
NW = 32
K = 128
CH = 80
EP = NW * CH * K
ROWS = NP // 16

_mesh = plsc.VectorSubcoreMesh(core_axis_name="c", subcore_axis_name="s")
_sc_params = pltpu.CompilerParams(use_tc_tiling_on_sc=False)


@functools.partial(
    pl.kernel,
    out_type=jax.ShapeDtypeStruct((2, NP, 8), jnp.float32),
    mesh=_mesh,
    scratch_types=[
        pltpu.VMEM((CH, K), jnp.int32),
        pltpu.VMEM((K, 8), jnp.float32),
        pltpu.VMEM_SHARED((NP, 8), jnp.float32),
    ],
    compiler_params=_sc_params,
)
def _deg_kernel(dst_hbm, ones_hbm, z8_hbm, out_hbm, idx_v, ones_v, acc_sh):
    c = lax.axis_index("c")
    s = lax.axis_index("s")
    w = 2 * s + c
    pltpu.sync_copy(z8_hbm.at[pl.ds(s * ROWS, ROWS)], acc_sh.at[pl.ds(s * ROWS, ROWS)])
    pltpu.sync_copy(dst_hbm.at[w], idx_v)
    pltpu.sync_copy(ones_hbm, ones_v)
    plsc.subcore_barrier()

    def body(j, carry):
        pltpu.sync_copy(ones_v, acc_sh.at[idx_v.at[j]], add=True)
        return carry

    lax.fori_loop(0, CH, body, 0)
    plsc.subcore_barrier()
    pltpu.sync_copy(acc_sh.at[pl.ds(s * ROWS, ROWS)], out_hbm.at[c, pl.ds(s * ROWS, ROWS)])


def _make_agg(d, nh):
    @functools.partial(
        pl.kernel,
        out_type=[jax.ShapeDtypeStruct((2, NP, d), jnp.float32) for _ in range(nh)],
        mesh=_mesh,
        scratch_types=[
            pltpu.VMEM((CH, K), jnp.int32),
            pltpu.VMEM((CH, K), jnp.int32),
            pltpu.VMEM((K, d), jnp.float32),
            pltpu.VMEM((K, d), jnp.float32),
            pltpu.VMEM_SHARED((NP, d), jnp.float32),
            pltpu.SemaphoreType.DMA,
            pltpu.SemaphoreType.DMA,
        ],
        compiler_params=_sc_params,
    )
    def agg(*args):
        ys = args[:nh]
        src_hbm, dst_hbm, zd_hbm = args[nh:nh + 3]
        outs = args[nh + 3:2 * nh + 3]
        src_v, dst_v, buf0, buf1, acc_sh, sem0, sem1 = args[2 * nh + 3:]
        c = lax.axis_index("c")
        s = lax.axis_index("s")
        w = 2 * s + c
        sl = pl.ds(s * ROWS, ROWS)
        pltpu.sync_copy(src_hbm.at[w], src_v)
        pltpu.sync_copy(dst_hbm.at[w], dst_v)
        bufs = (buf0, buf1)
        sems = (sem0, sem1)

        for g in range(nh):
            y_hbm = ys[g]
            pltpu.sync_copy(zd_hbm.at[sl], acc_sh.at[sl])
            plsc.subcore_barrier()
            pltpu.async_copy(y_hbm.at[src_v.at[0]], buf0, sem0)

            def body(t, carry):
                for b in (0, 1):
                    j = 2 * t + b
                    pltpu.make_async_copy(y_hbm.at[src_v.at[j]], bufs[b], sems[b]).wait()
                    jn = j + 1

                    @pl.when(jn < CH)
                    def _():
                        pltpu.async_copy(y_hbm.at[src_v.at[jn]], bufs[1 - b], sems[1 - b])

                    pltpu.sync_copy(bufs[b], acc_sh.at[dst_v.at[j]], add=True)
                return carry

            lax.fori_loop(0, CH // 2, body, 0)
            plsc.subcore_barrier()
            pltpu.sync_copy(acc_sh.at[sl], outs[g].at[c, sl])

    return agg


_agg64x2 = _make_agg(D_IN // 2, 2)
_agg16 = _make_agg(NCLS, 1)


def _scale_body(p_ref, x_ref, dinv_ref, y1a_ref, y1b_ref):
    counts = p_ref[0] + p_ref[1]
    deg = counts[:, :1] + 1.0
    row = lax.broadcasted_iota(jnp.int32, (NP, 1), 0)
    dinv = jnp.where(row < N, lax.rsqrt(deg), 0.0)
    dinv_ref[...] = dinv
    y1a_ref[...] = x_ref[:, : D_IN // 2] * dinv
    y1b_ref[...] = x_ref[:, D_IN // 2 :] * dinv


def _mlp_body(dinv_ref, sa_ref, sb_ref, y1a_ref, y1b_ref,
              w1_ref, b1_ref, w2_ref, y2_ref):
    dinv = dinv_ref[...]
    agg = jnp.concatenate(
        [sa_ref[0] + sa_ref[1] + y1a_ref[...],
         sb_ref[0] + sb_ref[1] + y1b_ref[...]], axis=1) * dinv
    h = jnp.dot(agg, w1_ref[...], preferred_element_type=jnp.float32)
    h = jnp.maximum(h + b1_ref[...], 0.0)
    z = jnp.dot(h, w2_ref[...], preferred_element_type=jnp.float32)
    y2_ref[...] = z * dinv


def _out_body(dinv_ref, sp_ref, y2_ref, b2_ref, o_ref):
    o_ref[...] = ((sp_ref[0] + sp_ref[1] + y2_ref[...]) * dinv_ref[...]
                  + b2_ref[...])


def kernel(x, edge_index, W1, b1, W2, b2):
    ei = edge_index.astype(jnp.int32)
    pad = N + (jnp.arange(EP - E, dtype=jnp.int32) % (NP - N))
    srcp = jnp.concatenate([ei[0], pad]).reshape(NW, CH, K)
    dstp = jnp.concatenate([ei[1], pad]).reshape(NW, CH, K)
    xp = jnp.pad(x, ((0, NP - N), (0, 0)))

    ones = jnp.ones((K, 8), jnp.float32)
    z8 = jnp.zeros((NP, 8), jnp.float32)
    z16 = jnp.zeros((NP, NCLS), jnp.float32)
    z64 = jnp.zeros((NP, D_IN // 2), jnp.float32)

    deg_p = _deg_kernel(dstp, ones, z8)

    dinv, y1a, y1b = pl.pallas_call(
        _scale_body,
        out_shape=[
            jax.ShapeDtypeStruct((NP, 1), jnp.float32),
            jax.ShapeDtypeStruct((NP, D_IN // 2), jnp.float32),
            jax.ShapeDtypeStruct((NP, D_IN // 2), jnp.float32),
        ],
    )(deg_p, xp)

    s1a, s1b = _agg64x2(y1a, y1b, srcp, dstp, z64)

    y2 = pl.pallas_call(
        _mlp_body,
        out_shape=jax.ShapeDtypeStruct((NP, NCLS), jnp.float32),
    )(dinv, s1a, s1b, y1a, y1b, W1, b1.reshape(1, HID), W2)

    (s2,) = _agg16(y2, srcp, dstp, z16)

    outp = pl.pallas_call(
        _out_body,
        out_shape=jax.ShapeDtypeStruct((NP, NCLS), jnp.float32),
    )(dinv, s2, y2, b2.reshape(1, NCLS))

    return outp[:N]


# speedup vs baseline: 37.2017x; 1.2822x over previous
import functools

import jax
import jax.numpy as jnp
from jax import lax
from jax.experimental import pallas as pl
from jax.experimental.pallas import tpu as pltpu
from jax.experimental.pallas import tpu_sc as plsc

N = 10000
NP = 10240
D_IN = 128
HID = 256
NCLS = 16
E = 320000
NW = 32
K = 128
CH = 80
EP = NW * CH * K
ROWS = NP // 16

_mesh = plsc.VectorSubcoreMesh(core_axis_name="c", subcore_axis_name="s")
_sc_params = pltpu.CompilerParams(use_tc_tiling_on_sc=False)


@functools.partial(
    pl.kernel,
    out_type=jax.ShapeDtypeStruct((2, NP, 8), jnp.float32),
    mesh=_mesh,
    scratch_types=[
        pltpu.VMEM((CH, K), jnp.int32),
        pltpu.VMEM((K, 8), jnp.float32),
        pltpu.VMEM_SHARED((NP, 8), jnp.float32),
        pltpu.SemaphoreType.DMA,
        pltpu.SemaphoreType.DMA,
        pltpu.SemaphoreType.DMA,
        pltpu.SemaphoreType.DMA,
    ],
    compiler_params=_sc_params,
)
def _deg_kernel(dst_hbm, ones_hbm, z8_hbm, out_hbm, idx_v, ones_v, acc_sh,
                *sems):
    c = lax.axis_index("c")
    s = lax.axis_index("s")
    w = 2 * s + c
    pltpu.sync_copy(z8_hbm.at[pl.ds(s * ROWS, ROWS)], acc_sh.at[pl.ds(s * ROWS, ROWS)])
    pltpu.sync_copy(dst_hbm.at[w], idx_v)
    pltpu.sync_copy(ones_hbm, ones_v)
    plsc.subcore_barrier()

    def body(t, carry):
        for b in range(4):
            j = 4 * t + b

            @pl.when(j >= 4)
            def _():
                pltpu.make_async_copy(
                    ones_v, acc_sh.at[idx_v.at[j - 4]], sems[b]).wait()

            pltpu.async_copy(ones_v, acc_sh.at[idx_v.at[j]], sems[b], add=True)
        return carry

    lax.fori_loop(0, CH // 4, body, 0)
    for b in range(4):
        pltpu.make_async_copy(ones_v, acc_sh.at[idx_v.at[CH - 4 + b]], sems[b]).wait()
    plsc.subcore_barrier()
    pltpu.sync_copy(acc_sh.at[pl.ds(s * ROWS, ROWS)], out_hbm.at[c, pl.ds(s * ROWS, ROWS)])


def _make_agg(d, nh):
    NB = 4

    @functools.partial(
        pl.kernel,
        out_type=[jax.ShapeDtypeStruct((2, NP, d), jnp.float32) for _ in range(nh)],
        mesh=_mesh,
        scratch_types=[
            pltpu.VMEM((CH, K), jnp.int32),
            pltpu.VMEM((CH, K), jnp.int32),
            [pltpu.VMEM((K, d), jnp.float32) for _ in range(NB)],
            [pltpu.SemaphoreType.DMA for _ in range(NB)],
            [pltpu.SemaphoreType.DMA for _ in range(NB)],
            pltpu.VMEM_SHARED((NP, d), jnp.float32),
        ],
        compiler_params=_sc_params,
    )
    def agg(*args):
        ys = args[:nh]
        src_hbm, dst_hbm, zd_hbm = args[nh:nh + 3]
        outs = args[nh + 3:2 * nh + 3]
        src_v, dst_v, bufs, gsems, ssems, acc_sh = args[2 * nh + 3:]
        c = lax.axis_index("c")
        s = lax.axis_index("s")
        w = 2 * s + c
        sl = pl.ds(s * ROWS, ROWS)
        pltpu.sync_copy(src_hbm.at[w], src_v)
        pltpu.sync_copy(dst_hbm.at[w], dst_v)

        G = NB // 2

        for g in range(nh):
            y_hbm = ys[g]
            pltpu.sync_copy(zd_hbm.at[sl], acc_sh.at[sl])
            plsc.subcore_barrier()
            for j0 in range(G):
                pltpu.async_copy(y_hbm.at[src_v.at[j0]], bufs[j0], gsems[j0])

            def body(t, carry):
                for b in range(NB):
                    j = NB * t + b
                    pltpu.make_async_copy(
                        y_hbm.at[src_v.at[j]], bufs[b], gsems[b]).wait()
                    pltpu.async_copy(
                        bufs[b], acc_sh.at[dst_v.at[j]], ssems[b], add=True)
                    jg = j + G

                    @pl.when(jg < CH)
                    def _():
                        bg = (b + G) % NB
                        @pl.when(j >= NB - G)
                        def _():
                            pltpu.make_async_copy(
                                bufs[bg], acc_sh.at[dst_v.at[jg - NB]],
                                ssems[bg]).wait()

                        pltpu.async_copy(
                            y_hbm.at[src_v.at[jg]], bufs[bg], gsems[bg])
                return carry

            lax.fori_loop(0, CH // NB, body, 0)
            for b in range(NB):
                pltpu.make_async_copy(
                    bufs[b], acc_sh.at[dst_v.at[CH - NB + b]], ssems[b]).wait()

            plsc.subcore_barrier()
            pltpu.sync_copy(acc_sh.at[sl], outs[g].at[c, sl])

    return agg


_agg64x2 = _make_agg(D_IN // 2, 2)
_agg16 = _make_agg(NCLS, 1)


def _scale_body(p_ref, x_ref, dinv_ref, y1a_ref, y1b_ref):
    counts = p_ref[0] + p_ref[1]
    deg = counts[:, :1] + 1.0
    row = lax.broadcasted_iota(jnp.int32, (NP, 1), 0)
    dinv = jnp.where(row < N, lax.rsqrt(deg), 0.0)
    dinv_ref[...] = dinv
    y1a_ref[...] = x_ref[:, : D_IN // 2] * dinv
    y1b_ref[...] = x_ref[:, D_IN // 2 :] * dinv


def _mlp_body(dinv_ref, sa_ref, sb_ref, y1a_ref, y1b_ref,
              w1_ref, b1_ref, w2_ref, y2_ref):
    dinv = dinv_ref[...]
    agg = jnp.concatenate(
        [sa_ref[0] + sa_ref[1] + y1a_ref[...],
         sb_ref[0] + sb_ref[1] + y1b_ref[...]], axis=1) * dinv
    h = jnp.dot(agg, w1_ref[...], preferred_element_type=jnp.float32)
    h = jnp.maximum(h + b1_ref[...], 0.0)
    z = jnp.dot(h, w2_ref[...], preferred_element_type=jnp.float32)
    y2_ref[...] = z * dinv


def _out_body(dinv_ref, sp_ref, y2_ref, b2_ref, o_ref):
    o_ref[...] = ((sp_ref[0] + sp_ref[1] + y2_ref[...]) * dinv_ref[...]
                  + b2_ref[...])


def kernel(x, edge_index, W1, b1, W2, b2):
    ei = edge_index.astype(jnp.int32)
    pad = N + (jnp.arange(EP - E, dtype=jnp.int32) % (NP - N))
    srcp = jnp.concatenate([ei[0], pad]).reshape(NW, CH, K)
    dstp = jnp.concatenate([ei[1], pad]).reshape(NW, CH, K)
    xp = jnp.pad(x, ((0, NP - N), (0, 0)))

    ones = jnp.ones((K, 8), jnp.float32)
    z8 = jnp.zeros((NP, 8), jnp.float32)
    z16 = jnp.zeros((NP, NCLS), jnp.float32)
    z64 = jnp.zeros((NP, D_IN // 2), jnp.float32)

    deg_p = _deg_kernel(dstp, ones, z8)

    dinv, y1a, y1b = pl.pallas_call(
        _scale_body,
        out_shape=[
            jax.ShapeDtypeStruct((NP, 1), jnp.float32),
            jax.ShapeDtypeStruct((NP, D_IN // 2), jnp.float32),
            jax.ShapeDtypeStruct((NP, D_IN // 2), jnp.float32),
        ],
    )(deg_p, xp)

    s1a, s1b = _agg64x2(y1a, y1b, srcp, dstp, z64)

    y2 = pl.pallas_call(
        _mlp_body,
        out_shape=jax.ShapeDtypeStruct((NP, NCLS), jnp.float32),
    )(dinv, s1a, s1b, y1a, y1b, W1, b1.reshape(1, HID), W2)

    (s2,) = _agg16(y2, srcp, dstp, z16)

    outp = pl.pallas_call(
        _out_body,
        out_shape=jax.ShapeDtypeStruct((NP, NCLS), jnp.float32),
    )(dinv, s2, y2, b2.reshape(1, NCLS))

    return outp[:N]


# speedup vs baseline: 44.9567x; 1.2085x over previous
import functools

import jax
import jax.numpy as jnp
from jax import lax
from jax.experimental import pallas as pl
from jax.experimental.pallas import tpu as pltpu
from jax.experimental.pallas import tpu_sc as plsc

N = 10000
NP = 10240
D_IN = 128
HID = 256
NCLS = 16
E = 320000
NW = 32
K = 128
CH = 80
EP = NW * CH * K
ROWS = NP // 16
BM = 1024

_mesh = plsc.VectorSubcoreMesh(core_axis_name="c", subcore_axis_name="s")
_sc_params = pltpu.CompilerParams(use_tc_tiling_on_sc=False)


@functools.partial(
    pl.kernel,
    out_type=jax.ShapeDtypeStruct((2, NP, 8), jnp.float32),
    mesh=_mesh,
    scratch_types=[
        pltpu.VMEM((CH, K), jnp.int32),
        pltpu.VMEM((K, 8), jnp.float32),
        pltpu.VMEM_SHARED((NP, 8), jnp.float32),
        pltpu.SemaphoreType.DMA,
        pltpu.SemaphoreType.DMA,
        pltpu.SemaphoreType.DMA,
        pltpu.SemaphoreType.DMA,
    ],
    compiler_params=_sc_params,
)
def _deg_kernel(dst_hbm, ones_hbm, z8_hbm, out_hbm, idx_v, ones_v, acc_sh,
                *sems):
    c = lax.axis_index("c")
    s = lax.axis_index("s")
    w = 2 * s + c
    sl = pl.ds(s * ROWS, ROWS)
    pltpu.sync_copy(z8_hbm.at[sl], acc_sh.at[sl])
    pltpu.sync_copy(dst_hbm.at[w], idx_v)
    pltpu.sync_copy(ones_hbm, ones_v)
    plsc.subcore_barrier()

    def body(t, carry):
        for b in range(4):
            j = 4 * t + b

            @pl.when(j >= 4)
            def _():
                pltpu.make_async_copy(
                    ones_v, acc_sh.at[idx_v.at[j - 4]], sems[b]).wait()

            pltpu.async_copy(ones_v, acc_sh.at[idx_v.at[j]], sems[b], add=True)
        return carry

    lax.fori_loop(0, CH // 4, body, 0)
    for b in range(4):
        pltpu.make_async_copy(ones_v, acc_sh.at[idx_v.at[CH - 4 + b]], sems[b]).wait()
    plsc.subcore_barrier()
    pltpu.sync_copy(acc_sh.at[sl], out_hbm.at[c, sl])


def _make_agg(d, nh):
    NB = 8
    G = NB // 2

    @functools.partial(
        pl.kernel,
        out_type=jax.ShapeDtypeStruct((2, NP, nh * d), jnp.float32),
        mesh=_mesh,
        scratch_types=[
            pltpu.VMEM((CH, K), jnp.int32),
            pltpu.VMEM((CH, K), jnp.int32),
            [pltpu.VMEM((K, d), jnp.float32) for _ in range(NB)],
            [pltpu.SemaphoreType.DMA for _ in range(NB)],
            [pltpu.SemaphoreType.DMA for _ in range(NB)],
            pltpu.VMEM_SHARED((NP, d), jnp.float32),
        ],
        compiler_params=_sc_params,
    )
    def agg(y_hbm, src_hbm, dst_hbm, zd_hbm, out_hbm,
            src_v, dst_v, bufs, gsems, ssems, acc_sh):
        c = lax.axis_index("c")
        s = lax.axis_index("s")
        w = 2 * s + c
        sl = pl.ds(s * ROWS, ROWS)
        pltpu.sync_copy(dst_hbm.at[w], dst_v)

        for g in range(nh):
            pltpu.sync_copy(src_hbm.at[g, w], src_v)
            pltpu.sync_copy(zd_hbm.at[sl], acc_sh.at[sl])
            plsc.subcore_barrier()
            for j0 in range(G):
                pltpu.async_copy(y_hbm.at[src_v.at[j0]], bufs[j0], gsems[j0])

            def body(t, carry):
                for b in range(NB):
                    j = NB * t + b
                    pltpu.make_async_copy(
                        y_hbm.at[src_v.at[j]], bufs[b], gsems[b]).wait()
                    pltpu.async_copy(
                        bufs[b], acc_sh.at[dst_v.at[j]], ssems[b], add=True)
                    jg = j + G

                    @pl.when(jg < CH)
                    def _():
                        bg = (b + G) % NB
                        @pl.when(j >= NB - G)
                        def _():
                            pltpu.make_async_copy(
                                bufs[bg], acc_sh.at[dst_v.at[jg - NB]],
                                ssems[bg]).wait()

                        pltpu.async_copy(
                            y_hbm.at[src_v.at[jg]], bufs[bg], gsems[bg])
                return carry

            lax.fori_loop(0, CH // NB, body, 0)
            for b in range(NB):
                pltpu.make_async_copy(
                    bufs[b], acc_sh.at[dst_v.at[CH - NB + b]], ssems[b]).wait()

            plsc.subcore_barrier()
            pltpu.sync_copy(acc_sh.at[sl], out_hbm.at[c, sl, pl.ds(g * d, d)])

    return agg


_agg64x2 = _make_agg(D_IN // 2, 2)
_agg16 = _make_agg(NCLS, 1)


def _scale_body(p_ref, x_ref, dinv_ref, y1_ref):
    counts = p_ref[0] + p_ref[1]
    deg = counts[:, :1] + 1.0
    i = pl.program_id(0)
    row = i * BM + lax.broadcasted_iota(jnp.int32, (BM, 1), 0)
    dinv = jnp.where(row < N, lax.rsqrt(deg), 0.0)
    dinv_ref[...] = dinv
    y1_ref[...] = x_ref[...] * dinv


def _mlp_body(dinv_ref, sp_ref, y1_ref, w1_ref, b1_ref, w2_ref, y2_ref):
    dinv = dinv_ref[...]
    agg = (sp_ref[0] + sp_ref[1] + y1_ref[...]) * dinv
    h = jnp.dot(agg, w1_ref[...], preferred_element_type=jnp.float32)
    h = jnp.maximum(h + b1_ref[...], 0.0)
    z = jnp.dot(h, w2_ref[...], preferred_element_type=jnp.float32)
    y2_ref[...] = z * dinv


def _out_body(dinv_ref, sp_ref, y2_ref, b2_ref, o_ref):
    o_ref[...] = ((sp_ref[0] + sp_ref[1] + y2_ref[...]) * dinv_ref[...]
                  + b2_ref[...])


_scale_call = pl.pallas_call(
    _scale_body,
    grid=(NP // BM,),
    in_specs=[
        pl.BlockSpec((2, BM, 8), lambda i: (0, i, 0)),
        pl.BlockSpec((BM, D_IN), lambda i: (i, 0)),
    ],
    out_specs=[
        pl.BlockSpec((BM, 1), lambda i: (i, 0)),
        pl.BlockSpec((BM, D_IN), lambda i: (i, 0)),
    ],
    out_shape=[
        jax.ShapeDtypeStruct((NP, 1), jnp.float32),
        jax.ShapeDtypeStruct((NP, D_IN), jnp.float32),
    ],
)

_mlp_call = pl.pallas_call(
    _mlp_body,
    grid=(NP // BM,),
    in_specs=[
        pl.BlockSpec((BM, 1), lambda i: (i, 0)),
        pl.BlockSpec((2, BM, D_IN), lambda i: (0, i, 0)),
        pl.BlockSpec((BM, D_IN), lambda i: (i, 0)),
        pl.BlockSpec((D_IN, HID), lambda i: (0, 0)),
        pl.BlockSpec((1, HID), lambda i: (0, 0)),
        pl.BlockSpec((HID, NCLS), lambda i: (0, 0)),
    ],
    out_specs=pl.BlockSpec((BM, NCLS), lambda i: (i, 0)),
    out_shape=jax.ShapeDtypeStruct((NP, NCLS), jnp.float32),
)

_out_call = pl.pallas_call(
    _out_body,
    out_shape=jax.ShapeDtypeStruct((NP, NCLS), jnp.float32),
)


def kernel(x, edge_index, W1, b1, W2, b2):
    ei = edge_index.astype(jnp.int32)
    pad = N + (jnp.arange(EP - E, dtype=jnp.int32) % (NP - N))
    srcp = jnp.concatenate([ei[0], pad]).reshape(NW, CH, K)
    dstp = jnp.concatenate([ei[1], pad]).reshape(NW, CH, K)
    srcp2 = jnp.stack([2 * srcp, 2 * srcp + 1])
    xp = jnp.pad(x, ((0, NP - N), (0, 0)))

    ones = jnp.ones((K, 8), jnp.float32)
    z8 = jnp.zeros((NP, 8), jnp.float32)
    z16 = jnp.zeros((NP, NCLS), jnp.float32)
    z64 = jnp.zeros((NP, D_IN // 2), jnp.float32)

    deg_p = _deg_kernel(dstp, ones, z8)

    dinv, y1 = _scale_call(deg_p, xp)

    s1 = _agg64x2(y1.reshape(2 * NP, D_IN // 2), srcp2, dstp, z64)

    y2 = _mlp_call(dinv, s1, y1, W1, b1.reshape(1, HID), W2)

    s2 = _agg16(y2, srcp.reshape(1, NW, CH, K), dstp, z16)

    outp = _out_call(dinv, s2, y2, b2.reshape(1, NCLS))

    return outp[:N]


# speedup vs baseline: 47.2967x; 1.0520x over previous
import functools

import jax
import jax.numpy as jnp
from jax import lax
from jax.experimental import pallas as pl
from jax.experimental.pallas import tpu as pltpu
from jax.experimental.pallas import tpu_sc as plsc

N = 10000
NP = 10240
D_IN = 128
HID = 256
NCLS = 16
E = 320000
NW = 32
K = 128
CH = 80
EP = NW * CH * K
ROWS = NP // 16
BM = 1024

_mesh = plsc.VectorSubcoreMesh(core_axis_name="c", subcore_axis_name="s")
_sc_params = pltpu.CompilerParams(use_tc_tiling_on_sc=False,
                                  needs_layout_passes=False)


@functools.partial(
    pl.kernel,
    out_type=jax.ShapeDtypeStruct((2, NP // 128, 128), jnp.float32),
    mesh=_mesh,
    scratch_types=[
        pltpu.VMEM((CH, K), jnp.int32),
        pltpu.VMEM((K, 8), jnp.float32),
        pltpu.VMEM((ROWS, 8), jnp.float32),
        pltpu.VMEM((ROWS // 128, 128), jnp.float32),
        pltpu.VMEM_SHARED((NP, 8), jnp.float32),
        pltpu.SemaphoreType.DMA,
        pltpu.SemaphoreType.DMA,
        pltpu.SemaphoreType.DMA,
        pltpu.SemaphoreType.DMA,
    ],
    compiler_params=_sc_params,
)
def _deg_kernel(dst_hbm, ones_hbm, z8_hbm, out_hbm, idx_v, ones_v, dbuf, cbuf,
                acc_sh, *sems):
    c = lax.axis_index("c")
    s = lax.axis_index("s")
    w = 2 * s + c
    sl = pl.ds(s * ROWS, ROWS)
    pltpu.sync_copy(z8_hbm.at[sl], acc_sh.at[sl])
    pltpu.sync_copy(dst_hbm.at[w], idx_v)
    pltpu.sync_copy(ones_hbm, ones_v)
    plsc.subcore_barrier()

    def body(t, carry):
        for b in range(4):
            j = 4 * t + b

            @pl.when(j >= 4)
            def _():
                pltpu.make_async_copy(
                    ones_v, acc_sh.at[idx_v.at[j - 4]], sems[b]).wait()

            pltpu.async_copy(ones_v, acc_sh.at[idx_v.at[j]], sems[b], add=True)
        return carry

    lax.fori_loop(0, CH // 4, body, 0)
    for b in range(4):
        pltpu.make_async_copy(ones_v, acc_sh.at[idx_v.at[CH - 4 + b]], sems[b]).wait()
    plsc.subcore_barrier()
    pltpu.sync_copy(acc_sh.at[sl], dbuf)
    zero16 = jnp.zeros((16,), jnp.int32)
    iota16 = lax.iota(jnp.int32, 16)
    for t in range(ROWS // 16):
        v = plsc.load_gather(dbuf, [iota16 + 16 * t, zero16])
        cbuf[t // 8, pl.ds((t % 8) * 16, 16)] = v
    pltpu.sync_copy(cbuf, out_hbm.at[c, pl.ds(s * (ROWS // 128), ROWS // 128)])


def _make_agg(d, nh):
    NB = 8
    G = NB // 2

    @functools.partial(
        pl.kernel,
        out_type=jax.ShapeDtypeStruct((2, NP, nh * d), jnp.float32),
        mesh=_mesh,
        scratch_types=[
            pltpu.VMEM((CH, K), jnp.int32),
            pltpu.VMEM((CH, K), jnp.int32),
            [pltpu.VMEM((K, d), jnp.float32) for _ in range(NB)],
            [pltpu.SemaphoreType.DMA for _ in range(NB)],
            [pltpu.SemaphoreType.DMA for _ in range(NB)],
            pltpu.VMEM_SHARED((NP, d), jnp.float32),
        ],
        compiler_params=_sc_params,
    )
    def agg(y_hbm, src_hbm, dst_hbm, zd_hbm, out_hbm,
            src_v, dst_v, bufs, gsems, ssems, acc_sh):
        c = lax.axis_index("c")
        s = lax.axis_index("s")
        w = 2 * s + c
        sl = pl.ds(s * ROWS, ROWS)
        pltpu.sync_copy(dst_hbm.at[w], dst_v)

        for g in range(nh):
            pltpu.sync_copy(src_hbm.at[g, w], src_v)
            pltpu.sync_copy(zd_hbm.at[sl], acc_sh.at[sl])
            plsc.subcore_barrier()
            for j0 in range(G):
                pltpu.async_copy(y_hbm.at[src_v.at[j0]], bufs[j0], gsems[j0])

            def body(t, carry):
                for b in range(NB):
                    j = NB * t + b
                    pltpu.make_async_copy(
                        y_hbm.at[src_v.at[j]], bufs[b], gsems[b]).wait()
                    pltpu.async_copy(
                        bufs[b], acc_sh.at[dst_v.at[j]], ssems[b], add=True)
                    jg = j + G

                    @pl.when(jg < CH)
                    def _():
                        bg = (b + G) % NB
                        @pl.when(j >= NB - G)
                        def _():
                            pltpu.make_async_copy(
                                bufs[bg], acc_sh.at[dst_v.at[jg - NB]],
                                ssems[bg]).wait()

                        pltpu.async_copy(
                            y_hbm.at[src_v.at[jg]], bufs[bg], gsems[bg])
                return carry

            lax.fori_loop(0, CH // NB, body, 0)
            for b in range(NB):
                pltpu.make_async_copy(
                    bufs[b], acc_sh.at[dst_v.at[CH - NB + b]], ssems[b]).wait()

            plsc.subcore_barrier()
            pltpu.sync_copy(acc_sh.at[sl], out_hbm.at[c, sl, pl.ds(g * d, d)])

    return agg


_agg64x2 = _make_agg(D_IN // 2, 2)
_agg16 = _make_agg(NCLS, 1)


def _expand(dinv_c):
    rows = 128 * dinv_c.shape[0]
    sel = (lax.broadcasted_iota(jnp.int32, (rows, dinv_c.shape[0]), 0) // 128
           == lax.broadcasted_iota(jnp.int32, (rows, dinv_c.shape[0]), 1))
    full = jnp.dot(sel.astype(jnp.float32), dinv_c,
                   preferred_element_type=jnp.float32)
    cm = (lax.broadcasted_iota(jnp.int32, (rows, 128), 1)
          == lax.broadcasted_iota(jnp.int32, (rows, 128), 0) % 128)
    return jnp.sum(jnp.where(cm, full, 0.0), axis=1, keepdims=True)


def _scale_body(p_ref, x_ref, dinvc_ref, y1_ref):
    deg = p_ref[0] + p_ref[1] + 1.0
    i = pl.program_id(0)
    nid = (i * BM
           + 128 * lax.broadcasted_iota(jnp.int32, (BM // 128, 128), 0)
           + lax.broadcasted_iota(jnp.int32, (BM // 128, 128), 1))
    dinv_c = jnp.where(nid < N, lax.rsqrt(deg), 0.0)
    dinvc_ref[...] = dinv_c
    y1_ref[...] = x_ref[...] * _expand(dinv_c)


def _mlp_body(dinvc_ref, sp_ref, y1_ref, w1_ref, b1_ref, w2_ref, y2_ref):
    dinv = _expand(dinvc_ref[...])
    agg = (sp_ref[0] + sp_ref[1] + y1_ref[...]) * dinv
    h = jnp.dot(agg, w1_ref[...], preferred_element_type=jnp.float32)
    h = jnp.maximum(h + b1_ref[...], 0.0)
    z = jnp.dot(h, w2_ref[...], preferred_element_type=jnp.float32)
    y2_ref[...] = z * dinv


def _out_body(dinvc_ref, sp_ref, y2_ref, b2_ref, o_ref):
    dinv = _expand(dinvc_ref[...])
    o_ref[...] = (sp_ref[0] + sp_ref[1] + y2_ref[...]) * dinv + b2_ref[...]


_scale_call = pl.pallas_call(
    _scale_body,
    grid=(NP // BM,),
    in_specs=[
        pl.BlockSpec((2, BM // 128, 128), lambda i: (0, i, 0)),
        pl.BlockSpec((BM, D_IN), lambda i: (i, 0)),
    ],
    out_specs=[
        pl.BlockSpec((BM // 128, 128), lambda i: (i, 0)),
        pl.BlockSpec((BM, D_IN), lambda i: (i, 0)),
    ],
    out_shape=[
        jax.ShapeDtypeStruct((NP // 128, 128), jnp.float32),
        jax.ShapeDtypeStruct((NP, D_IN), jnp.float32),
    ],
)

_mlp_call = pl.pallas_call(
    _mlp_body,
    grid=(NP // BM,),
    in_specs=[
        pl.BlockSpec((BM // 128, 128), lambda i: (i, 0)),
        pl.BlockSpec((2, BM, D_IN), lambda i: (0, i, 0)),
        pl.BlockSpec((BM, D_IN), lambda i: (i, 0)),
        pl.BlockSpec((D_IN, HID), lambda i: (0, 0)),
        pl.BlockSpec((1, HID), lambda i: (0, 0)),
        pl.BlockSpec((HID, NCLS), lambda i: (0, 0)),
    ],
    out_specs=pl.BlockSpec((BM, NCLS), lambda i: (i, 0)),
    out_shape=jax.ShapeDtypeStruct((NP, NCLS), jnp.float32),
)

_out_call = pl.pallas_call(
    _out_body,
    out_shape=jax.ShapeDtypeStruct((NP, NCLS), jnp.float32),
)


def kernel(x, edge_index, W1, b1, W2, b2):
    ei = edge_index.astype(jnp.int32)
    pad = N + (jnp.arange(EP - E, dtype=jnp.int32) % (NP - N))
    srcp = jnp.concatenate([ei[0], pad]).reshape(NW, CH, K)
    dstp = jnp.concatenate([ei[1], pad]).reshape(NW, CH, K)
    srcp2 = jnp.stack([2 * srcp, 2 * srcp + 1])
    xp = jnp.pad(x, ((0, NP - N), (0, 0)))

    ones = jnp.ones((K, 8), jnp.float32)
    z8 = jnp.zeros((NP, 8), jnp.float32)
    z16 = jnp.zeros((NP, NCLS), jnp.float32)
    z64 = jnp.zeros((NP, D_IN // 2), jnp.float32)

    deg_p = _deg_kernel(dstp, ones, z8)

    dinv_c, y1 = _scale_call(deg_p, xp)

    s1 = _agg64x2(y1.reshape(2 * NP, D_IN // 2), srcp2, dstp, z64)

    y2 = _mlp_call(dinv_c, s1, y1, W1, b1.reshape(1, HID), W2)

    s2 = _agg16(y2, srcp.reshape(1, NW, CH, K), dstp, z16)

    outp = _out_call(dinv_c, s2, y2, b2.reshape(1, NCLS))

    return outp[:N]
